# baseline (speedup 1.0000x reference)
import functools

import jax
import jax.numpy as jnp
from jax import lax
from jax.experimental import pallas as pl
from jax.experimental.pallas import tpu as pltpu
from jax.experimental.pallas import tpu_sc as plsc

N = 10000
NP = 10112
NE = 320000
NW = 32
CH = 64
NCK = 158
NCK_A = NCK + 2
EPW_A = NCK_A * CH
RPT = NP // 16
RB = 2528
GRID = NP // RB


def _mm(x, w):
    return lax.dot_general(x, w, (((1,), (1,)), ((), ())),
                           preferred_element_type=jnp.float32)


def _lrelu(x):
    return jnp.where(x >= 0, x, 0.2 * x)


def _rff_body(pos_ref, b_ref, o_ref):
    vp = (2.0 * jnp.pi) * _mm(pos_ref[...], b_ref[...])
    o_ref[...] = jnp.concatenate([jnp.cos(vp), jnp.sin(vp)], axis=-1)


def _rff(pos_p, B):
    return pl.pallas_call(
        _rff_body,
        grid=(GRID,),
        in_specs=[pl.BlockSpec((RB, 3), lambda i: (i, 0)),
                  pl.BlockSpec((64, 3), lambda i: (0, 0))],
        out_specs=pl.BlockSpec((RB, 128), lambda i: (i, 0)),
        out_shape=jax.ShapeDtypeStruct((NP, 128), jnp.float32),
    )(pos_p, B)


def _pre_body(x_ref, p_ref, w1, b1, w2, b2, wg, wx, bf, a_ref, c_ref):
    x = x_ref[...]
    t = jnp.maximum(_mm(x, w1[...]) + b1[...], 0.0)
    delta = jnp.tanh(_mm(t, w2[...]) + b2[...])
    pg = _mm(p_ref[...], wg[...])
    a_ref[...] = pg + _mm(x, wx[...])
    c_ref[...] = _mm(delta, wg[...]) - pg + bf[...]


def _pre(x, p, bp):
    wf = bp["f"]["W"]
    full = lambda s: pl.BlockSpec(s, lambda i: tuple(0 for _ in s))
    return pl.pallas_call(
        _pre_body,
        grid=(GRID,),
        in_specs=[pl.BlockSpec((RB, 128), lambda i: (i, 0)),
                  pl.BlockSpec((RB, 3), lambda i: (i, 0)),
                  full((128, 128)), full((1, 128)),
                  full((3, 128)), full((1, 3)),
                  full((128, 3)), full((128, 128)), full((1, 128))],
        out_specs=[pl.BlockSpec((RB, 128), lambda i: (i, 0)),
                   pl.BlockSpec((RB, 128), lambda i: (i, 0))],
        out_shape=[jax.ShapeDtypeStruct((NP, 128), jnp.float32),
                   jax.ShapeDtypeStruct((NP, 128), jnp.float32)],
    )(x, p,
      bp["h1"]["W"], bp["h1"]["b"].reshape(1, 128),
      bp["h2"]["W"], bp["h2"]["b"].reshape(1, 3),
      wf[:, :3], wf[:, 3:], bp["f"]["b"].reshape(1, 128))


def _post_body(pp_ref, x_ref, wg1, bg1, wg2, bg2, o_ref):
    pp = pp_ref[...]
    aggr = pp[0] + pp[1]
    u = jnp.maximum(_mm(aggr, wg1[...]) + bg1[...], 0.0)
    o_ref[...] = x_ref[...] + jnp.maximum(_mm(u, wg2[...]) + bg2[...], 0.0)


def _post(partial, x, bp):
    full = lambda s: pl.BlockSpec(s, lambda i: tuple(0 for _ in s))
    return pl.pallas_call(
        _post_body,
        grid=(GRID,),
        in_specs=[pl.BlockSpec((2, RB, 128), lambda i: (0, i, 0)),
                  pl.BlockSpec((RB, 128), lambda i: (i, 0)),
                  full((128, 128)), full((1, 128)),
                  full((128, 128)), full((1, 128))],
        out_specs=pl.BlockSpec((RB, 128), lambda i: (i, 0)),
        out_shape=jax.ShapeDtypeStruct((NP, 128), jnp.float32),
    )(partial, x,
      bp["g1"]["W"], bp["g1"]["b"].reshape(1, 128),
      bp["g2"]["W"], bp["g2"]["b"].reshape(1, 128))


def _max_body(x_ref, o_ref):
    m = jnp.max(x_ref[...], axis=0, keepdims=True)

    @pl.when(pl.program_id(0) == 0)
    def _init():
        o_ref[...] = m

    @pl.when(pl.program_id(0) > 0)
    def _acc():
        o_ref[...] = jnp.maximum(o_ref[...], m)


def _max10k(x):
    return pl.pallas_call(
        _max_body,
        grid=(5,),
        in_specs=[pl.BlockSpec((2000, 128), lambda i: (i, 0))],
        out_specs=pl.BlockSpec((1, 128), lambda i: (0, 0)),
        out_shape=jax.ShapeDtypeStruct((1, 128), jnp.float32),
    )(x)


def _mid_body(x_ref, h_ref, wpg, bpg, wt1x, wt1h, bt1, wt2, bt2,
              wgx, wgh, bgc, p2_ref, x2_ref):
    hp = _lrelu(_mm(h_ref[...], wpg[...]) + bpg[...])
    c1 = _mm(hp, wt1h[...]) + bt1[...]
    cg = _mm(hp, wgh[...]) + bgc[...]
    x = x_ref[...]
    t1 = _lrelu(_mm(x, wt1x[...]) + c1)
    p2_ref[...] = jnp.tanh(_mm(t1, wt2[...]) + bt2[...])
    x2_ref[...] = _lrelu(_mm(x, wgx[...]) + cg)


def _mid(x, h, P):
    full = lambda s: pl.BlockSpec(s, lambda i: tuple(0 for _ in s))
    wt1 = P["tail1"]["W"]
    wgc = P["gg_gc"]["W"]
    return pl.pallas_call(
        _mid_body,
        grid=(GRID,),
        in_specs=[pl.BlockSpec((RB, 128), lambda i: (i, 0)),
                  full((1, 128)),
                  full((128, 128)), full((1, 128)),
                  full((64, 128)), full((64, 128)), full((1, 64)),
                  full((3, 64)), full((1, 3)),
                  full((128, 128)), full((128, 128)), full((1, 128))],
        out_specs=[pl.BlockSpec((RB, 3), lambda i: (i, 0)),
                   pl.BlockSpec((RB, 128), lambda i: (i, 0))],
        out_shape=[jax.ShapeDtypeStruct((NP, 3), jnp.float32),
                   jax.ShapeDtypeStruct((NP, 128), jnp.float32)],
    )(x, h,
      P["pg_gc"]["W"], P["pg_gc"]["b"].reshape(1, 128),
      wt1[:, :128], wt1[:, 128:], P["tail1"]["b"].reshape(1, 64),
      P["tail2"]["W"], P["tail2"]["b"].reshape(1, 3),
      wgc[:, :128], wgc[:, 128:], P["gg_gc"]["b"].reshape(1, 128))


_sc_mesh = plsc.VectorSubcoreMesh(core_axis_name="c", subcore_axis_name="s")


_COPY_CHUNKS = [(k * CH, CH) for k in range(9)] + [(9 * CH, 56)]


@functools.partial(
    pl.kernel,
    mesh=_sc_mesh,
    out_type=jax.ShapeDtypeStruct((2, NP, 128), jnp.float32),
    scratch_types=[
        pltpu.VMEM((CH,), jnp.int32),
        pltpu.VMEM((CH,), jnp.int32),
        pltpu.VMEM((CH,), jnp.int32),
        pltpu.VMEM((CH,), jnp.int32),
        pltpu.VMEM((CH,), jnp.int32),
        pltpu.VMEM((CH,), jnp.int32),
        pltpu.VMEM((CH, 128), jnp.float32),
        pltpu.VMEM((CH, 128), jnp.float32),
        pltpu.VMEM((CH, 128), jnp.float32),
        pltpu.VMEM((CH, 128), jnp.float32),
        pltpu.VMEM((CH, 128), jnp.float32),
        pltpu.VMEM((CH, 128), jnp.float32),
        pltpu.VMEM_SHARED((NP, 128), jnp.float32),
        pltpu.SemaphoreType.DMA,
        pltpu.SemaphoreType.DMA,
        pltpu.SemaphoreType.DMA,
        pltpu.SemaphoreType.DMA,
        pltpu.SemaphoreType.DMA,
        pltpu.SemaphoreType.DMA,
        pltpu.SemaphoreType.DMA,
        pltpu.SemaphoreType.DMA,
        pltpu.SemaphoreType.DMA,
        pltpu.SemaphoreType.DMA,
        pltpu.SemaphoreType.DMA,
        pltpu.SemaphoreType.DMA,
        pltpu.SemaphoreType.DMA,
        pltpu.SemaphoreType.DMA,
        pltpu.SemaphoreType.DMA,
    ],
)
def _edge_kernel(a_hbm, c_hbm, src_hbm, dst_hbm, out_hbm,
                 si0, si1, si2, di0, di1, di2,
                 ab0, ab1, ab2, cb0, cb1, cb2, acc,
                 ssi0, ssi1, ssi2, sdi0, sdi1, sdi2,
                 sga0, sga1, sga2, sgc0, sgc1, sgc2, ssc0, ssc1, ssc2):
    cid = lax.axis_index("c")
    sid = lax.axis_index("s")
    wid = sid * 2 + cid
    base = wid * EPW_A
    slots = [(si0, di0, ab0, cb0, ssi0, sdi0, sga0, sgc0, ssc0),
             (si1, di1, ab1, cb1, ssi1, sdi1, sga1, sgc1, ssc1),
             (si2, di2, ab2, cb2, ssi2, sdi2, sga2, sgc2, ssc2)]

    def _issue_idx(j, s):
        si, di, _, _, ssi, sdi, _, _, _ = slots[s]
        off = base + j * CH
        pltpu.async_copy(src_hbm.at[pl.ds(off, CH)], si, ssi)
        pltpu.async_copy(dst_hbm.at[pl.ds(off, CH)], di, sdi)

    def _wait_idx(s):
        si, di, _, _, ssi, sdi, _, _, _ = slots[s]
        pltpu.make_async_copy(src_hbm.at[pl.ds(base, CH)], si, ssi).wait()
        pltpu.make_async_copy(dst_hbm.at[pl.ds(base, CH)], di, sdi).wait()

    def _issue_gathers(s):
        si, di, ab, cb, _, _, sga, sgc, _ = slots[s]
        pltpu.async_copy(a_hbm.at[si], ab, sga)
        pltpu.async_copy(c_hbm.at[di], cb, sgc)

    def _process(s):
        si, di, ab, cb, _, _, sga, sgc, ssc = slots[s]
        pltpu.make_async_copy(a_hbm.at[si], ab, sga).wait()
        pltpu.make_async_copy(c_hbm.at[di], cb, sgc).wait()

        def _relu_add(e, cc):
            for u in range(2):
                for v in range(8):
                    sl = pl.ds(v * 16, 16)
                    ab[2 * e + u, sl] = jnp.maximum(
                        ab[2 * e + u, sl] + cb[2 * e + u, sl], 0.0)
            return cc
        lax.fori_loop(0, CH // 2, _relu_add, 0)
        pltpu.async_copy(ab, acc.at[di], ssc, add=True)

    def _wait_scatter(s):
        _, di, ab, _, _, _, _, _, ssc = slots[s]
        pltpu.make_async_copy(ab, acc.at[di], ssc).wait()

    _issue_idx(0, 0)
    _issue_idx(1, 1)

    def _zero(e, carry):
        for v in range(8):
            ab2[e, pl.ds(v * 16, 16)] = jnp.zeros((16,), jnp.float32)
        return carry
    lax.fori_loop(0, CH, _zero, 0)
    row0 = sid * RPT
    for r, nr in _COPY_CHUNKS:
        pltpu.sync_copy(ab2.at[pl.ds(0, nr)], acc.at[pl.ds(row0 + r, nr)])

    _wait_idx(0)
    _issue_gathers(0)
    _wait_idx(1)
    _issue_gathers(1)
    plsc.subcore_barrier()

    _process(0)
    _issue_idx(2, 2)
    _wait_idx(2)
    _issue_gathers(2)
    _process(1)
    _wait_scatter(0)
    _issue_idx(3, 0)

    def _third(j, s_proc, s_next, s_pre):
        _wait_idx(s_next)
        _issue_gathers(s_next)
        _process(s_proc)
        _wait_scatter(s_pre)
        _issue_idx(j + 2, s_pre)

    def _steady(it, carry):
        g = 2 + it * 3
        _third(g, 2, 0, 1)
        _third(g + 1, 0, 1, 2)
        _third(g + 2, 1, 2, 0)
        return carry
    lax.fori_loop(0, (NCK - 2) // 3, _steady, 0)

    _wait_scatter(1)
    pltpu.make_async_copy(a_hbm.at[si2], ab2, sga2).wait()
    pltpu.make_async_copy(c_hbm.at[di2], cb2, sgc2).wait()
    _wait_idx(0)

    plsc.subcore_barrier()
    for r, nr in _COPY_CHUNKS:
        pltpu.sync_copy(acc.at[pl.ds(row0 + r, nr)], ab0.at[pl.ds(0, nr)])
        pltpu.sync_copy(ab0.at[pl.ds(0, nr)],
                        out_hbm.at[cid, pl.ds(row0 + r, nr)])


def _conv(bp, x, p, src_p, dst_p):
    a, c = _pre(x, p, bp)
    partial = _edge_kernel(a, c, src_p, dst_p)
    return _post(partial, x, bp)


def kernel(pos, edge_index, batch, params):
    pos_p = jnp.zeros((NP, 3), jnp.float32).at[:N].set(pos)
    pad = NW * NCK * CH - NE
    rng = jnp.arange(pad, dtype=jnp.int32)
    srcb = jnp.concatenate([edge_index[0], rng % N]).reshape(NW, NCK * CH)
    dstb = jnp.concatenate([edge_index[1],
                            N + (rng % (NP - N))]).reshape(NW, NCK * CH)
    dum = (NCK_A - NCK) * CH
    src_p = jnp.concatenate(
        [srcb, jnp.zeros((NW, dum), jnp.int32)], axis=1).reshape(-1)
    dst_p = jnp.concatenate(
        [dstb, jnp.full((NW, dum), N, jnp.int32)], axis=1).reshape(-1)
    P = params
    x = _rff(pos_p, P["B"])
    x = _conv(P["conv1"], x, pos_p, src_p, dst_p)
    x = _conv(P["conv2"], x, pos_p, src_p, dst_p)
    h = _max10k(x)
    pos2, x2 = _mid(x, h, P)
    x2 = _conv(P["block1"], x2, pos2, src_p, dst_p)
    x2 = _conv(P["block2"], x2, pos2, src_p, dst_p)
    return (x2[:N], pos2[:N])


# speedup vs baseline: 1.0292x; 1.0292x over previous
import functools

import jax
import jax.numpy as jnp
from jax import lax
from jax.experimental import pallas as pl
from jax.experimental.pallas import tpu as pltpu
from jax.experimental.pallas import tpu_sc as plsc

N = 10000
NP = 10112
NE = 320000
NW = 32
CH = 64
NCK = 158
NCK_A = NCK + 2
EPW_A = NCK_A * CH
RPT = NP // 16
RB = 2528
GRID = NP // RB


def _mm(x, w):
    return lax.dot_general(x, w, (((1,), (1,)), ((), ())),
                           preferred_element_type=jnp.float32)


def _lrelu(x):
    return jnp.where(x >= 0, x, 0.2 * x)


def _full(s):
    return pl.BlockSpec(s, lambda i: tuple(0 for _ in s))


def _row(d):
    return pl.BlockSpec((RB, d), lambda i: (i, 0))


def _pre_args(bp):
    wf = bp["f"]["W"]
    return (bp["h1"]["W"], bp["h1"]["b"].reshape(1, 128),
            bp["h2"]["W"], bp["h2"]["b"].reshape(1, 3),
            wf[:, :3], wf[:, 3:], bp["f"]["b"].reshape(1, 128))


_PRE_SPECS = [_full((128, 128)), _full((1, 128)),
              _full((3, 128)), _full((1, 3)),
              _full((128, 3)), _full((128, 128)), _full((1, 128))]


def _pre_compute(x, p, w1, b1, w2, b2, wg, wx, bf):
    t = jnp.maximum(_mm(x, w1[...]) + b1[...], 0.0)
    delta = jnp.tanh(_mm(t, w2[...]) + b2[...])
    pg = _mm(p, wg[...])
    a = pg + _mm(x, wx[...])
    c = _mm(delta, wg[...]) - pg + bf[...]
    return a, c


def _stage1_body(pos_ref, b_ref, w1, b1, w2, b2, wg, wx, bf,
                 x_ref, a_ref, c_ref):
    pos = pos_ref[...]
    vp = (2.0 * jnp.pi) * _mm(pos, b_ref[...])
    x = jnp.concatenate([jnp.cos(vp), jnp.sin(vp)], axis=-1)
    x_ref[...] = x
    a_ref[...], c_ref[...] = _pre_compute(x, pos, w1, b1, w2, b2, wg, wx, bf)


def _stage1(pos_p, P):
    return pl.pallas_call(
        _stage1_body,
        grid=(GRID,),
        in_specs=[_row(3), _full((64, 3))] + _PRE_SPECS,
        out_specs=[_row(128), _row(128), _row(128)],
        out_shape=[jax.ShapeDtypeStruct((NP, 128), jnp.float32)] * 3,
    )(pos_p, P["B"], *_pre_args(P["conv1"]))


def _postpre_body(pp_ref, x_ref, p_ref, wg1, bg1, wg2, bg2,
                  w1, b1, w2, b2, wg, wx, bf, xo_ref, a_ref, c_ref):
    pp = pp_ref[...]
    aggr = pp[0] + pp[1]
    u = jnp.maximum(_mm(aggr, wg1[...]) + bg1[...], 0.0)
    xn = x_ref[...] + jnp.maximum(_mm(u, wg2[...]) + bg2[...], 0.0)
    xo_ref[...] = xn
    a_ref[...], c_ref[...] = _pre_compute(xn, p_ref[...],
                                          w1, b1, w2, b2, wg, wx, bf)


def _postpre(partial, x, p, bp, bp_next):
    return pl.pallas_call(
        _postpre_body,
        grid=(GRID,),
        in_specs=[pl.BlockSpec((2, RB, 128), lambda i: (0, i, 0)),
                  _row(128), _row(3),
                  _full((128, 128)), _full((1, 128)),
                  _full((128, 128)), _full((1, 128))] + _PRE_SPECS,
        out_specs=[_row(128), _row(128), _row(128)],
        out_shape=[jax.ShapeDtypeStruct((NP, 128), jnp.float32)] * 3,
    )(partial, x, p,
      bp["g1"]["W"], bp["g1"]["b"].reshape(1, 128),
      bp["g2"]["W"], bp["g2"]["b"].reshape(1, 128),
      *_pre_args(bp_next))


def _post_body(pp_ref, x_ref, wg1, bg1, wg2, bg2, o_ref):
    pp = pp_ref[...]
    aggr = pp[0] + pp[1]
    u = jnp.maximum(_mm(aggr, wg1[...]) + bg1[...], 0.0)
    o_ref[...] = x_ref[...] + jnp.maximum(_mm(u, wg2[...]) + bg2[...], 0.0)


def _post(partial, x, bp):
    return pl.pallas_call(
        _post_body,
        grid=(GRID,),
        in_specs=[pl.BlockSpec((2, RB, 128), lambda i: (0, i, 0)),
                  _row(128),
                  _full((128, 128)), _full((1, 128)),
                  _full((128, 128)), _full((1, 128))],
        out_specs=_row(128),
        out_shape=jax.ShapeDtypeStruct((NP, 128), jnp.float32),
    )(partial, x,
      bp["g1"]["W"], bp["g1"]["b"].reshape(1, 128),
      bp["g2"]["W"], bp["g2"]["b"].reshape(1, 128))


def _max_body(x_ref, o_ref):
    m = jnp.max(x_ref[...], axis=0, keepdims=True)

    @pl.when(pl.program_id(0) == 0)
    def _init():
        o_ref[...] = m

    @pl.when(pl.program_id(0) > 0)
    def _acc():
        o_ref[...] = jnp.maximum(o_ref[...], m)


def _max10k(x):
    return pl.pallas_call(
        _max_body,
        grid=(5,),
        in_specs=[pl.BlockSpec((2000, 128), lambda i: (i, 0))],
        out_specs=pl.BlockSpec((1, 128), lambda i: (0, 0)),
        out_shape=jax.ShapeDtypeStruct((1, 128), jnp.float32),
    )(x)


def _mid_body(x_ref, h_ref, wpg, bpg, wt1x, wt1h, bt1, wt2, bt2,
              wgx, wgh, bgc, w1, b1, w2, b2, wg, wx, bf,
              p2_ref, y_ref, a_ref, c_ref):
    hp = _lrelu(_mm(h_ref[...], wpg[...]) + bpg[...])
    c1 = _mm(hp, wt1h[...]) + bt1[...]
    cg = _mm(hp, wgh[...]) + bgc[...]
    x = x_ref[...]
    t1 = _lrelu(_mm(x, wt1x[...]) + c1)
    p2 = jnp.tanh(_mm(t1, wt2[...]) + bt2[...])
    y = _lrelu(_mm(x, wgx[...]) + cg)
    p2_ref[...] = p2
    y_ref[...] = y
    a_ref[...], c_ref[...] = _pre_compute(y, p2, w1, b1, w2, b2, wg, wx, bf)


def _mid(x, h, P):
    wt1 = P["tail1"]["W"]
    wgc = P["gg_gc"]["W"]
    return pl.pallas_call(
        _mid_body,
        grid=(GRID,),
        in_specs=[_row(128), _full((1, 128)),
                  _full((128, 128)), _full((1, 128)),
                  _full((64, 128)), _full((64, 128)), _full((1, 64)),
                  _full((3, 64)), _full((1, 3)),
                  _full((128, 128)), _full((128, 128)),
                  _full((1, 128))] + _PRE_SPECS,
        out_specs=[_row(3), _row(128), _row(128), _row(128)],
        out_shape=[jax.ShapeDtypeStruct((NP, 3), jnp.float32),
                   jax.ShapeDtypeStruct((NP, 128), jnp.float32),
                   jax.ShapeDtypeStruct((NP, 128), jnp.float32),
                   jax.ShapeDtypeStruct((NP, 128), jnp.float32)],
    )(x, h,
      P["pg_gc"]["W"], P["pg_gc"]["b"].reshape(1, 128),
      wt1[:, :128], wt1[:, 128:], P["tail1"]["b"].reshape(1, 64),
      P["tail2"]["W"], P["tail2"]["b"].reshape(1, 3),
      wgc[:, :128], wgc[:, 128:], P["gg_gc"]["b"].reshape(1, 128),
      *_pre_args(P["block1"]))


_sc_mesh = plsc.VectorSubcoreMesh(core_axis_name="c", subcore_axis_name="s")


_COPY_CHUNKS = [(k * CH, CH) for k in range(9)] + [(9 * CH, 56)]


@functools.partial(
    pl.kernel,
    mesh=_sc_mesh,
    out_type=jax.ShapeDtypeStruct((2, NP, 128), jnp.float32),
    scratch_types=[
        pltpu.VMEM((CH,), jnp.int32),
        pltpu.VMEM((CH,), jnp.int32),
        pltpu.VMEM((CH,), jnp.int32),
        pltpu.VMEM((CH,), jnp.int32),
        pltpu.VMEM((CH,), jnp.int32),
        pltpu.VMEM((CH,), jnp.int32),
        pltpu.VMEM((CH, 128), jnp.float32),
        pltpu.VMEM((CH, 128), jnp.float32),
        pltpu.VMEM((CH, 128), jnp.float32),
        pltpu.VMEM((CH, 128), jnp.float32),
        pltpu.VMEM((CH, 128), jnp.float32),
        pltpu.VMEM((CH, 128), jnp.float32),
        pltpu.VMEM_SHARED((NP, 128), jnp.float32),
        pltpu.SemaphoreType.DMA,
        pltpu.SemaphoreType.DMA,
        pltpu.SemaphoreType.DMA,
        pltpu.SemaphoreType.DMA,
        pltpu.SemaphoreType.DMA,
        pltpu.SemaphoreType.DMA,
        pltpu.SemaphoreType.DMA,
        pltpu.SemaphoreType.DMA,
        pltpu.SemaphoreType.DMA,
        pltpu.SemaphoreType.DMA,
        pltpu.SemaphoreType.DMA,
        pltpu.SemaphoreType.DMA,
        pltpu.SemaphoreType.DMA,
        pltpu.SemaphoreType.DMA,
        pltpu.SemaphoreType.DMA,
    ],
)
def _edge_kernel(a_hbm, c_hbm, src_hbm, dst_hbm, out_hbm,
                 si0, si1, si2, di0, di1, di2,
                 ab0, ab1, ab2, cb0, cb1, cb2, acc,
                 ssi0, ssi1, ssi2, sdi0, sdi1, sdi2,
                 sga0, sga1, sga2, sgc0, sgc1, sgc2, ssc0, ssc1, ssc2):
    cid = lax.axis_index("c")
    sid = lax.axis_index("s")
    wid = sid * 2 + cid
    base = wid * EPW_A
    slots = [(si0, di0, ab0, cb0, ssi0, sdi0, sga0, sgc0, ssc0),
             (si1, di1, ab1, cb1, ssi1, sdi1, sga1, sgc1, ssc1),
             (si2, di2, ab2, cb2, ssi2, sdi2, sga2, sgc2, ssc2)]

    def _issue_idx(j, s):
        si, di, _, _, ssi, sdi, _, _, _ = slots[s]
        off = base + j * CH
        pltpu.async_copy(src_hbm.at[pl.ds(off, CH)], si, ssi)
        pltpu.async_copy(dst_hbm.at[pl.ds(off, CH)], di, sdi)

    def _wait_idx(s):
        si, di, _, _, ssi, sdi, _, _, _ = slots[s]
        pltpu.make_async_copy(src_hbm.at[pl.ds(base, CH)], si, ssi).wait()
        pltpu.make_async_copy(dst_hbm.at[pl.ds(base, CH)], di, sdi).wait()

    def _issue_gathers(s):
        si, di, ab, cb, _, _, sga, sgc, _ = slots[s]
        pltpu.async_copy(a_hbm.at[si], ab, sga)
        pltpu.async_copy(c_hbm.at[di], cb, sgc)

    def _process(s):
        si, di, ab, cb, _, _, sga, sgc, ssc = slots[s]
        pltpu.make_async_copy(a_hbm.at[si], ab, sga).wait()
        pltpu.make_async_copy(c_hbm.at[di], cb, sgc).wait()

        def _relu_add(e, cc):
            for u in range(2):
                for v in range(8):
                    sl = pl.ds(v * 16, 16)
                    ab[2 * e + u, sl] = jnp.maximum(
                        ab[2 * e + u, sl] + cb[2 * e + u, sl], 0.0)
            return cc
        lax.fori_loop(0, CH // 2, _relu_add, 0)
        pltpu.async_copy(ab, acc.at[di], ssc, add=True)

    def _wait_scatter(s):
        _, di, ab, _, _, _, _, _, ssc = slots[s]
        pltpu.make_async_copy(ab, acc.at[di], ssc).wait()

    _issue_idx(0, 0)
    _issue_idx(1, 1)

    def _zero(e, carry):
        for v in range(8):
            ab2[e, pl.ds(v * 16, 16)] = jnp.zeros((16,), jnp.float32)
        return carry
    lax.fori_loop(0, CH, _zero, 0)
    row0 = sid * RPT
    for r, nr in _COPY_CHUNKS:
        pltpu.sync_copy(ab2.at[pl.ds(0, nr)], acc.at[pl.ds(row0 + r, nr)])

    _wait_idx(0)
    _issue_gathers(0)
    _wait_idx(1)
    _issue_gathers(1)
    plsc.subcore_barrier()

    _process(0)
    _issue_idx(2, 2)
    _wait_idx(2)
    _issue_gathers(2)
    _process(1)
    _wait_scatter(0)
    _issue_idx(3, 0)

    def _third(j, s_proc, s_next, s_pre):
        _wait_idx(s_next)
        _issue_gathers(s_next)
        _process(s_proc)
        _wait_scatter(s_pre)
        _issue_idx(j + 2, s_pre)

    def _steady(it, carry):
        g = 2 + it * 3
        _third(g, 2, 0, 1)
        _third(g + 1, 0, 1, 2)
        _third(g + 2, 1, 2, 0)
        return carry
    lax.fori_loop(0, (NCK - 2) // 3, _steady, 0)

    _wait_scatter(1)
    pltpu.make_async_copy(a_hbm.at[si2], ab2, sga2).wait()
    pltpu.make_async_copy(c_hbm.at[di2], cb2, sgc2).wait()
    _wait_idx(0)

    plsc.subcore_barrier()
    for r, nr in _COPY_CHUNKS:
        pltpu.sync_copy(acc.at[pl.ds(row0 + r, nr)], ab0.at[pl.ds(0, nr)])
        pltpu.sync_copy(ab0.at[pl.ds(0, nr)],
                        out_hbm.at[cid, pl.ds(row0 + r, nr)])


def kernel(pos, edge_index, batch, params):
    pos_p = jnp.zeros((NP, 3), jnp.float32).at[:N].set(pos)
    pad = NW * NCK * CH - NE
    rng = jnp.arange(pad, dtype=jnp.int32)
    srcb = jnp.concatenate([edge_index[0], rng % N]).reshape(NW, NCK * CH)
    dstb = jnp.concatenate([edge_index[1],
                            N + (rng % (NP - N))]).reshape(NW, NCK * CH)
    dum = (NCK_A - NCK) * CH
    src_p = jnp.concatenate(
        [srcb, jnp.zeros((NW, dum), jnp.int32)], axis=1).reshape(-1)
    dst_p = jnp.concatenate(
        [dstb, jnp.full((NW, dum), N, jnp.int32)], axis=1).reshape(-1)
    P = params
    x0, a, c = _stage1(pos_p, P)
    pt = _edge_kernel(a, c, src_p, dst_p)
    x1, a, c = _postpre(pt, x0, pos_p, P["conv1"], P["conv2"])
    pt = _edge_kernel(a, c, src_p, dst_p)
    x2 = _post(pt, x1, P["conv2"])
    h = _max10k(x2)
    pos2, y, a, c = _mid(x2, h, P)
    pt = _edge_kernel(a, c, src_p, dst_p)
    y1, a, c = _postpre(pt, y, pos2, P["block1"], P["block2"])
    pt = _edge_kernel(a, c, src_p, dst_p)
    yf = _post(pt, y1, P["block2"])
    return (yf[:N], pos2[:N])


# speedup vs baseline: 1.0419x; 1.0124x over previous
import functools

import jax
import jax.numpy as jnp
from jax import lax
from jax.experimental import pallas as pl
from jax.experimental.pallas import tpu as pltpu
from jax.experimental.pallas import tpu_sc as plsc

N = 10000
NP = 10112
NE = 320000
NW = 32
CH = 64
NCK = 158
NCK_A = NCK + 2
EPW_A = NCK_A * CH
RPT = NP // 16
RB = 2528
GRID = NP // RB


def _mm(x, w):
    return lax.dot_general(x, w, (((1,), (1,)), ((), ())),
                           preferred_element_type=jnp.float32)


def _lrelu(x):
    return jnp.where(x >= 0, x, 0.2 * x)


def _full(s):
    return pl.BlockSpec(s, lambda i: tuple(0 for _ in s))


def _row(d):
    return pl.BlockSpec((RB, d), lambda i: (i, 0))


def _pre_args(bp):
    wf = bp["f"]["W"]
    return (bp["h1"]["W"], bp["h1"]["b"].reshape(1, 128),
            bp["h2"]["W"], bp["h2"]["b"].reshape(1, 3),
            wf[:, :3], wf[:, 3:], bp["f"]["b"].reshape(1, 128))


_PRE_SPECS = [_full((128, 128)), _full((1, 128)),
              _full((3, 128)), _full((1, 3)),
              _full((128, 3)), _full((128, 128)), _full((1, 128))]


def _pre_compute(x, p, w1, b1, w2, b2, wg, wx, bf):
    t = jnp.maximum(_mm(x, w1[...]) + b1[...], 0.0)
    delta = jnp.tanh(_mm(t, w2[...]) + b2[...])
    pg = _mm(p, wg[...])
    a = pg + _mm(x, wx[...])
    c = _mm(delta, wg[...]) - pg + bf[...]
    return a, c


def _stage1_body(pos_ref, b_ref, w1, b1, w2, b2, wg, wx, bf,
                 x_ref, a_ref, c_ref):
    pos = pos_ref[...]
    vp = (2.0 * jnp.pi) * _mm(pos, b_ref[...])
    x = jnp.concatenate([jnp.cos(vp), jnp.sin(vp)], axis=-1)
    x_ref[...] = x
    a_ref[...], c_ref[...] = _pre_compute(x, pos, w1, b1, w2, b2, wg, wx, bf)


def _stage1(pos_p, P):
    return pl.pallas_call(
        _stage1_body,
        grid=(GRID,),
        in_specs=[_row(3), _full((64, 3))] + _PRE_SPECS,
        out_specs=[_row(128), _row(128), _row(128)],
        out_shape=[jax.ShapeDtypeStruct((NP, 128), jnp.float32)] * 3,
    )(pos_p, P["B"], *_pre_args(P["conv1"]))


def _postpre_body(pp_ref, x_ref, p_ref, wg1, bg1, wg2, bg2,
                  w1, b1, w2, b2, wg, wx, bf, xo_ref, a_ref, c_ref):
    pp = pp_ref[...]
    aggr = pp[0] + pp[1]
    u = jnp.maximum(_mm(aggr, wg1[...]) + bg1[...], 0.0)
    xn = x_ref[...] + jnp.maximum(_mm(u, wg2[...]) + bg2[...], 0.0)
    xo_ref[...] = xn
    a_ref[...], c_ref[...] = _pre_compute(xn, p_ref[...],
                                          w1, b1, w2, b2, wg, wx, bf)


def _postpre(partial, x, p, bp, bp_next):
    return pl.pallas_call(
        _postpre_body,
        grid=(GRID,),
        in_specs=[pl.BlockSpec((2, RB, 128), lambda i: (0, i, 0)),
                  _row(128), _row(3),
                  _full((128, 128)), _full((1, 128)),
                  _full((128, 128)), _full((1, 128))] + _PRE_SPECS,
        out_specs=[_row(128), _row(128), _row(128)],
        out_shape=[jax.ShapeDtypeStruct((NP, 128), jnp.float32)] * 3,
    )(partial, x, p,
      bp["g1"]["W"], bp["g1"]["b"].reshape(1, 128),
      bp["g2"]["W"], bp["g2"]["b"].reshape(1, 128),
      *_pre_args(bp_next))


def _post_body(pp_ref, x_ref, wg1, bg1, wg2, bg2, o_ref):
    pp = pp_ref[...]
    aggr = pp[0] + pp[1]
    u = jnp.maximum(_mm(aggr, wg1[...]) + bg1[...], 0.0)
    o_ref[...] = x_ref[...] + jnp.maximum(_mm(u, wg2[...]) + bg2[...], 0.0)


def _post(partial, x, bp):
    return pl.pallas_call(
        _post_body,
        grid=(GRID,),
        in_specs=[pl.BlockSpec((2, RB, 128), lambda i: (0, i, 0)),
                  _row(128),
                  _full((128, 128)), _full((1, 128)),
                  _full((128, 128)), _full((1, 128))],
        out_specs=_row(128),
        out_shape=jax.ShapeDtypeStruct((NP, 128), jnp.float32),
    )(partial, x,
      bp["g1"]["W"], bp["g1"]["b"].reshape(1, 128),
      bp["g2"]["W"], bp["g2"]["b"].reshape(1, 128))


def _postmax_body(pp_ref, x_ref, wg1, bg1, wg2, bg2, o_ref, m_ref):
    pp = pp_ref[...]
    aggr = pp[0] + pp[1]
    u = jnp.maximum(_mm(aggr, wg1[...]) + bg1[...], 0.0)
    xn = x_ref[...] + jnp.maximum(_mm(u, wg2[...]) + bg2[...], 0.0)
    o_ref[...] = xn
    i = pl.program_id(0)
    row = i * RB + jax.lax.broadcasted_iota(jnp.int32, (RB, 128), 0)
    xm = jnp.where(row < N, xn, -jnp.inf)
    m = jnp.max(xm, axis=0, keepdims=True)

    @pl.when(i == 0)
    def _init():
        m_ref[...] = m

    @pl.when(i > 0)
    def _acc():
        m_ref[...] = jnp.maximum(m_ref[...], m)


def _postmax(partial, x, bp):
    return pl.pallas_call(
        _postmax_body,
        grid=(GRID,),
        in_specs=[pl.BlockSpec((2, RB, 128), lambda i: (0, i, 0)),
                  _row(128),
                  _full((128, 128)), _full((1, 128)),
                  _full((128, 128)), _full((1, 128))],
        out_specs=[_row(128), pl.BlockSpec((1, 128), lambda i: (0, 0))],
        out_shape=[jax.ShapeDtypeStruct((NP, 128), jnp.float32),
                   jax.ShapeDtypeStruct((1, 128), jnp.float32)],
    )(partial, x,
      bp["g1"]["W"], bp["g1"]["b"].reshape(1, 128),
      bp["g2"]["W"], bp["g2"]["b"].reshape(1, 128))


def _mid_body(x_ref, h_ref, wpg, bpg, wt1x, wt1h, bt1, wt2, bt2,
              wgx, wgh, bgc, w1, b1, w2, b2, wg, wx, bf,
              p2_ref, y_ref, a_ref, c_ref):
    hp = _lrelu(_mm(h_ref[...], wpg[...]) + bpg[...])
    c1 = _mm(hp, wt1h[...]) + bt1[...]
    cg = _mm(hp, wgh[...]) + bgc[...]
    x = x_ref[...]
    t1 = _lrelu(_mm(x, wt1x[...]) + c1)
    p2 = jnp.tanh(_mm(t1, wt2[...]) + bt2[...])
    y = _lrelu(_mm(x, wgx[...]) + cg)
    p2_ref[...] = p2
    y_ref[...] = y
    a_ref[...], c_ref[...] = _pre_compute(y, p2, w1, b1, w2, b2, wg, wx, bf)


def _mid(x, h, P):
    wt1 = P["tail1"]["W"]
    wgc = P["gg_gc"]["W"]
    return pl.pallas_call(
        _mid_body,
        grid=(GRID,),
        in_specs=[_row(128), _full((1, 128)),
                  _full((128, 128)), _full((1, 128)),
                  _full((64, 128)), _full((64, 128)), _full((1, 64)),
                  _full((3, 64)), _full((1, 3)),
                  _full((128, 128)), _full((128, 128)),
                  _full((1, 128))] + _PRE_SPECS,
        out_specs=[_row(3), _row(128), _row(128), _row(128)],
        out_shape=[jax.ShapeDtypeStruct((NP, 3), jnp.float32),
                   jax.ShapeDtypeStruct((NP, 128), jnp.float32),
                   jax.ShapeDtypeStruct((NP, 128), jnp.float32),
                   jax.ShapeDtypeStruct((NP, 128), jnp.float32)],
    )(x, h,
      P["pg_gc"]["W"], P["pg_gc"]["b"].reshape(1, 128),
      wt1[:, :128], wt1[:, 128:], P["tail1"]["b"].reshape(1, 64),
      P["tail2"]["W"], P["tail2"]["b"].reshape(1, 3),
      wgc[:, :128], wgc[:, 128:], P["gg_gc"]["b"].reshape(1, 128),
      *_pre_args(P["block1"]))


_sc_mesh = plsc.VectorSubcoreMesh(core_axis_name="c", subcore_axis_name="s")


_COPY_CHUNKS = [(k * CH, CH) for k in range(9)] + [(9 * CH, 56)]


@functools.partial(
    pl.kernel,
    mesh=_sc_mesh,
    out_type=jax.ShapeDtypeStruct((2, NP, 128), jnp.float32),
    scratch_types=[
        pltpu.VMEM((CH,), jnp.int32),
        pltpu.VMEM((CH,), jnp.int32),
        pltpu.VMEM((CH,), jnp.int32),
        pltpu.VMEM((CH,), jnp.int32),
        pltpu.VMEM((CH,), jnp.int32),
        pltpu.VMEM((CH,), jnp.int32),
        pltpu.VMEM((CH, 128), jnp.float32),
        pltpu.VMEM((CH, 128), jnp.float32),
        pltpu.VMEM((CH, 128), jnp.float32),
        pltpu.VMEM((CH, 128), jnp.float32),
        pltpu.VMEM((CH, 128), jnp.float32),
        pltpu.VMEM((CH, 128), jnp.float32),
        pltpu.VMEM_SHARED((NP, 128), jnp.float32),
        pltpu.SemaphoreType.DMA,
        pltpu.SemaphoreType.DMA,
        pltpu.SemaphoreType.DMA,
        pltpu.SemaphoreType.DMA,
        pltpu.SemaphoreType.DMA,
        pltpu.SemaphoreType.DMA,
        pltpu.SemaphoreType.DMA,
        pltpu.SemaphoreType.DMA,
        pltpu.SemaphoreType.DMA,
        pltpu.SemaphoreType.DMA,
        pltpu.SemaphoreType.DMA,
        pltpu.SemaphoreType.DMA,
        pltpu.SemaphoreType.DMA,
        pltpu.SemaphoreType.DMA,
        pltpu.SemaphoreType.DMA,
    ],
)
def _edge_kernel(a_hbm, c_hbm, src_hbm, dst_hbm, out_hbm,
                 si0, si1, si2, di0, di1, di2,
                 ab0, ab1, ab2, cb0, cb1, cb2, acc,
                 ssi0, ssi1, ssi2, sdi0, sdi1, sdi2,
                 sga0, sga1, sga2, sgc0, sgc1, sgc2, ssc0, ssc1, ssc2):
    cid = lax.axis_index("c")
    sid = lax.axis_index("s")
    wid = sid * 2 + cid
    base = wid * EPW_A
    slots = [(si0, di0, ab0, cb0, ssi0, sdi0, sga0, sgc0, ssc0),
             (si1, di1, ab1, cb1, ssi1, sdi1, sga1, sgc1, ssc1),
             (si2, di2, ab2, cb2, ssi2, sdi2, sga2, sgc2, ssc2)]

    def _issue_idx(j, s):
        si, di, _, _, ssi, sdi, _, _, _ = slots[s]
        off = base + j * CH
        pltpu.async_copy(src_hbm.at[pl.ds(off, CH)], si, ssi)
        pltpu.async_copy(dst_hbm.at[pl.ds(off, CH)], di, sdi)

    def _wait_idx(s):
        si, di, _, _, ssi, sdi, _, _, _ = slots[s]
        pltpu.make_async_copy(src_hbm.at[pl.ds(base, CH)], si, ssi).wait()
        pltpu.make_async_copy(dst_hbm.at[pl.ds(base, CH)], di, sdi).wait()

    def _issue_gathers(s):
        si, di, ab, cb, _, _, sga, sgc, _ = slots[s]
        pltpu.async_copy(a_hbm.at[si], ab, sga)
        pltpu.async_copy(c_hbm.at[di], cb, sgc)

    def _process(s):
        si, di, ab, cb, _, _, sga, sgc, ssc = slots[s]
        pltpu.make_async_copy(a_hbm.at[si], ab, sga).wait()
        pltpu.make_async_copy(c_hbm.at[di], cb, sgc).wait()

        def _relu_add(e, cc):
            for u in range(2):
                for v in range(8):
                    sl = pl.ds(v * 16, 16)
                    ab[2 * e + u, sl] = jnp.maximum(
                        ab[2 * e + u, sl] + cb[2 * e + u, sl], 0.0)
            return cc
        lax.fori_loop(0, CH // 2, _relu_add, 0)
        pltpu.async_copy(ab, acc.at[di], ssc, add=True)

    def _wait_scatter(s):
        _, di, ab, _, _, _, _, _, ssc = slots[s]
        pltpu.make_async_copy(ab, acc.at[di], ssc).wait()

    _issue_idx(0, 0)
    _issue_idx(1, 1)

    def _zero(e, carry):
        for v in range(8):
            ab2[e, pl.ds(v * 16, 16)] = jnp.zeros((16,), jnp.float32)
        return carry
    lax.fori_loop(0, CH, _zero, 0)
    row0 = sid * RPT
    for r, nr in _COPY_CHUNKS:
        pltpu.async_copy(ab2.at[pl.ds(0, nr)],
                         acc.at[pl.ds(row0 + r, nr)], ssc2)
    for r, nr in _COPY_CHUNKS:
        pltpu.make_async_copy(ab2.at[pl.ds(0, nr)],
                              acc.at[pl.ds(row0 + r, nr)], ssc2).wait()

    _wait_idx(0)
    _issue_gathers(0)
    _wait_idx(1)
    _issue_gathers(1)
    plsc.subcore_barrier()

    _process(0)
    _issue_idx(2, 2)
    _wait_idx(2)
    _issue_gathers(2)
    _process(1)
    _wait_scatter(0)
    _issue_idx(3, 0)

    def _third(j, s_proc, s_next, s_pre):
        _wait_idx(s_next)
        _issue_gathers(s_next)
        _process(s_proc)
        _wait_scatter(s_pre)
        _issue_idx(j + 2, s_pre)

    def _steady(it, carry):
        g = 2 + it * 3
        _third(g, 2, 0, 1)
        _third(g + 1, 0, 1, 2)
        _third(g + 2, 1, 2, 0)
        return carry
    lax.fori_loop(0, (NCK - 2) // 3, _steady, 0)

    _wait_scatter(1)
    pltpu.make_async_copy(a_hbm.at[si2], ab2, sga2).wait()
    pltpu.make_async_copy(c_hbm.at[di2], cb2, sgc2).wait()
    _wait_idx(0)

    plsc.subcore_barrier()
    bufs = (ab0, ab1)
    sin = (sga0, sga1)
    sout = (ssc0, ssc1)
    prev = [None, None]
    for k, (r, nr) in enumerate(_COPY_CHUNKS):
        b = k % 2
        if prev[b] is not None:
            pr, pnr = prev[b]
            pltpu.make_async_copy(
                bufs[b].at[pl.ds(0, pnr)],
                out_hbm.at[cid, pl.ds(row0 + pr, pnr)], sout[b]).wait()
        pltpu.async_copy(acc.at[pl.ds(row0 + r, nr)],
                         bufs[b].at[pl.ds(0, nr)], sin[b])
        pltpu.make_async_copy(acc.at[pl.ds(row0 + r, nr)],
                              bufs[b].at[pl.ds(0, nr)], sin[b]).wait()
        pltpu.async_copy(bufs[b].at[pl.ds(0, nr)],
                         out_hbm.at[cid, pl.ds(row0 + r, nr)], sout[b])
        prev[b] = (r, nr)
    for b in range(2):
        pr, pnr = prev[b]
        pltpu.make_async_copy(
            bufs[b].at[pl.ds(0, pnr)],
            out_hbm.at[cid, pl.ds(row0 + pr, pnr)], sout[b]).wait()


def kernel(pos, edge_index, batch, params):
    pos_p = jnp.zeros((NP, 3), jnp.float32).at[:N].set(pos)
    pad = NW * NCK * CH - NE
    rng = jnp.arange(pad, dtype=jnp.int32)
    srcb = jnp.concatenate([edge_index[0], rng % N]).reshape(NW, NCK * CH)
    dstb = jnp.concatenate([edge_index[1],
                            N + (rng % (NP - N))]).reshape(NW, NCK * CH)
    dum = (NCK_A - NCK) * CH
    src_p = jnp.concatenate(
        [srcb, jnp.zeros((NW, dum), jnp.int32)], axis=1).reshape(-1)
    dst_p = jnp.concatenate(
        [dstb, jnp.full((NW, dum), N, jnp.int32)], axis=1).reshape(-1)
    P = params
    x0, a, c = _stage1(pos_p, P)
    pt = _edge_kernel(a, c, src_p, dst_p)
    x1, a, c = _postpre(pt, x0, pos_p, P["conv1"], P["conv2"])
    pt = _edge_kernel(a, c, src_p, dst_p)
    x2, h = _postmax(pt, x1, P["conv2"])
    pos2, y, a, c = _mid(x2, h, P)
    pt = _edge_kernel(a, c, src_p, dst_p)
    y1, a, c = _postpre(pt, y, pos2, P["block1"], P["block2"])
    pt = _edge_kernel(a, c, src_p, dst_p)
    yf = _post(pt, y1, P["block2"])
    return (yf[:N], pos2[:N])
